# baseline (speedup 1.0000x reference)
import functools

import jax
import jax.numpy as jnp
from jax import lax
from jax.experimental import pallas as pl
from jax.experimental.pallas import tpu as pltpu
from jax.experimental.pallas import tpu_sc as plsc

D = 128
SEQ = 50
SEQ_PAD = 56
ENT_PER = 4
NBUF = 4
AHEAD = 2
NW = 32
NREP = 16


@functools.partial(jax.jit, static_argnames=("ent_per_w",))
def _sc_gather(table, idx, ent_per_w):
    n_streams = ent_per_w // ENT_PER
    n_outer = n_streams // NBUF
    assert n_outer * NBUF == n_streams
    idx_per_w = ent_per_w * SEQ_PAD
    mesh = plsc.VectorSubcoreMesh(core_axis_name="c", subcore_axis_name="s")

    @functools.partial(
        pl.kernel,
        mesh=mesh,
        out_type=jax.ShapeDtypeStruct((NW * ent_per_w, SEQ, D), jnp.float32),
        scratch_types=(
            [pltpu.VMEM((idx_per_w,), jnp.int32),
             pltpu.VMEM((NBUF, ENT_PER, SEQ_PAD, D), jnp.float32)]
            + [pltpu.SemaphoreType.DMA] * (2 * NBUF)
        ),
        compiler_params=pltpu.CompilerParams(use_tc_tiling_on_sc=True),
    )
    def k(table_hbm, idx_hbm, out_hbm, idx_v, rows_v, *sems):
        gsem = sems[:NBUF]
        osem = sems[NBUF:]
        wid = lax.axis_index("s") * 2 + lax.axis_index("c")
        e_base = wid * ent_per_w
        pltpu.sync_copy(idx_hbm.at[wid], idx_v)

        def gathers(s, slot):
            return [
                pltpu.make_async_copy(
                    table_hbm.at[idx_v.at[pl.ds(s * ENT_PER * SEQ_PAD
                                                + j * SEQ_PAD, SEQ_PAD)]],
                    rows_v.at[slot, j], gsem[slot])
                for j in range(ENT_PER)
            ]

        def out_copy(slot, s):
            return pltpu.make_async_copy(
                rows_v.at[slot].at[:, pl.ds(0, SEQ)],
                out_hbm.at[pl.ds(e_base + s * ENT_PER, ENT_PER)],
                osem[slot])

        for h in range(AHEAD):
            for c in gathers(h, h):
                c.start()

        def body(t, carry):
            for b in range(NBUF):
                s = t * NBUF + b
                sh = (b + AHEAD) % NBUF
                h = s + AHEAD

                @pl.when(h < n_streams)
                def _():
                    @pl.when(h >= NBUF)
                    def _():
                        out_copy(sh, 0).wait()
                    for c in gathers(h, sh):
                        c.start()

                for c in gathers(s, b):
                    c.wait()
                out_copy(b, s).start()
            return carry

        lax.fori_loop(0, n_outer, body, 0)

        for b in range(NBUF):
            out_copy(b, 0).wait()

    return k(table, idx)


def kernel(indices, embed_weight):
    bsz, seq = indices.shape
    v = embed_weight.shape[0]
    ent_per_w = bsz // NW
    table_rep = jnp.tile(embed_weight, (NREP, 1))
    idx_p = jnp.pad(indices.astype(jnp.int32), ((0, 0), (0, SEQ_PAD - seq)))
    idx_w = idx_p.reshape(NW, ent_per_w * SEQ_PAD)
    rep_off = (jnp.arange(NW, dtype=jnp.int32) % NREP * v)[:, None]
    return _sc_gather(table_rep, idx_w + rep_off, ent_per_w)
